# baseline (speedup 1.0000x reference)
import functools
import math

import jax
import jax.numpy as jnp
from jax import lax
from jax.experimental import pallas as pl
from jax.experimental.pallas import tpu as pltpu
from jax.experimental.pallas import tpu_sc as plsc

NC = 2
NS = 16
NW = NC * NS
LANES = 16
CHUNK = 128


def _sc_mesh():
    return plsc.VectorSubcoreMesh(
        core_axis_name="c", subcore_axis_name="s", num_cores=NC, num_subcores=NS
    )


def _sc_degree(dst_r, np_rows):
    nch = dst_r.shape[1]
    rpt = np_rows // NS

    @functools.partial(
        pl.kernel,
        out_type=jax.ShapeDtypeStruct((NC, np_rows), jnp.float32),
        mesh=_sc_mesh(),
        compiler_params=pltpu.CompilerParams(use_tc_tiling_on_sc=False),
        scratch_types=[
            pltpu.VMEM((nch, CHUNK), jnp.int32),
            pltpu.VMEM((CHUNK,), jnp.float32),
            pltpu.VMEM((rpt,), jnp.float32),
            pltpu.VMEM_SHARED((np_rows,), jnp.float32),
        ],
    )
    def k(dst_hbm, out_hbm, dst_v, ones_v, z_v, acc_sh):
        c = lax.axis_index("c")
        s = lax.axis_index("s")
        wid = c * NS + s
        pltpu.sync_copy(dst_hbm.at[wid], dst_v)
        for i in range(CHUNK // LANES):
            ones_v[pl.ds(i * LANES, LANES)] = jnp.full((LANES,), 1.0, jnp.float32)
        for i in range(rpt // LANES):
            z_v[pl.ds(i * LANES, LANES)] = jnp.zeros((LANES,), jnp.float32)
        pltpu.sync_copy(z_v, acc_sh.at[pl.ds(s * rpt, rpt)])
        plsc.subcore_barrier()

        def step(j, carry):
            pltpu.sync_copy(ones_v, acc_sh.at[dst_v.at[j]], add=True)
            return carry

        lax.fori_loop(0, nch, step, 0)
        plsc.subcore_barrier()
        pltpu.sync_copy(acc_sh.at[pl.ds(s * rpt, rpt)], out_hbm.at[c, pl.ds(s * rpt, rpt)])

    return k(dst_r)


NCH_ALL = 160
NCH_HALF = 80
NQA = 40


def _agg_loop(tab_dummy_hbm, tab_sh, src_hbm, dst_hbm, tile, nch, src_v, dst_v,
              bufs, acc_sh, gsem, ssem):
    nq4 = NQA // 4
    for seg in range(nch // NQA):
        pltpu.sync_copy(src_hbm.at[tile, pl.ds(seg * NQA, NQA)], src_v)
        pltpu.sync_copy(dst_hbm.at[tile, pl.ds(seg * NQA, NQA)], dst_v)
        pltpu.async_copy(tab_sh.at[src_v.at[0]], bufs[0], gsem)
        pltpu.async_copy(tab_sh.at[src_v.at[1]], bufs[1], gsem)

        def body(j4, carry):
            j0 = j4 * 4
            for b in range(4):
                j = j0 + b
                nb = (b + 2) % 4
                if b < 2:
                    @pl.when(j4 > 0)
                    def _w():
                        pltpu.make_async_copy(
                            tab_dummy_hbm.at[pl.ds(0, CHUNK)], bufs[nb], ssem
                        ).wait()
                else:
                    pltpu.make_async_copy(
                        tab_dummy_hbm.at[pl.ds(0, CHUNK)], bufs[nb], ssem
                    ).wait()
                if b < 2:
                    pltpu.async_copy(tab_sh.at[src_v.at[j + 2]], bufs[nb], gsem)
                else:
                    @pl.when(j4 < nq4 - 1)
                    def _f():
                        pltpu.async_copy(tab_sh.at[src_v.at[j + 2]], bufs[nb], gsem)
                pltpu.make_async_copy(
                    tab_dummy_hbm.at[pl.ds(0, CHUNK)], bufs[b], gsem
                ).wait()
                pltpu.async_copy(bufs[b], acc_sh.at[dst_v.at[j]], ssem, add=True)
            return carry

        lax.fori_loop(0, nq4, body, 0)
        pltpu.make_async_copy(tab_dummy_hbm.at[pl.ds(0, CHUNK)], bufs[2], ssem).wait()
        pltpu.make_async_copy(tab_dummy_hbm.at[pl.ds(0, CHUNK)], bufs[3], ssem).wait()


def _sc_agg_chsplit(tab2, src_r, dst_r, np_rows, d):
    assert src_r.shape == (NS, NCH_ALL, CHUNK)
    rpt = np_rows // NS

    @functools.partial(
        pl.kernel,
        out_type=jax.ShapeDtypeStruct((NC, np_rows, d), jnp.float32),
        mesh=_sc_mesh(),
        compiler_params=pltpu.CompilerParams(use_tc_tiling_on_sc=False),
        scratch_types=[
            pltpu.VMEM((NQA, CHUNK), jnp.int32),
            pltpu.VMEM((NQA, CHUNK), jnp.int32),
            pltpu.VMEM((CHUNK, d), jnp.float32),
            pltpu.VMEM((CHUNK, d), jnp.float32),
            pltpu.VMEM((CHUNK, d), jnp.float32),
            pltpu.VMEM((CHUNK, d), jnp.float32),
            pltpu.VMEM_SHARED((np_rows, d), jnp.float32),
            pltpu.VMEM_SHARED((np_rows, d), jnp.float32),
            pltpu.SemaphoreType.DMA,
            pltpu.SemaphoreType.DMA,
        ],
    )
    def k(tab_hbm, src_hbm, dst_hbm, out_hbm, src_v, dst_v, r0, r1, r2, r3,
          tab_sh, acc_sh, gsem, ssem):
        c = lax.axis_index("c")
        s = lax.axis_index("s")
        sl = pl.ds(s * rpt, rpt)
        pltpu.sync_copy(tab_hbm.at[c, sl], tab_sh.at[sl])
        pltpu.sync_copy(tab_hbm.at[c, sl], acc_sh.at[sl])
        plsc.subcore_barrier()
        _agg_loop(tab_hbm.at[0], tab_sh, src_hbm, dst_hbm, s, NCH_ALL,
                  src_v, dst_v, [r0, r1, r2, r3], acc_sh, gsem, ssem)
        plsc.subcore_barrier()
        pltpu.sync_copy(acc_sh.at[sl], out_hbm.at[c, sl])

    return k(tab2, src_r, dst_r)


def _sc_agg_edgesplit(table, src_r, dst_r, np_rows, d):
    assert src_r.shape == (NW, NCH_HALF, CHUNK)
    rpt = np_rows // NS

    @functools.partial(
        pl.kernel,
        out_type=jax.ShapeDtypeStruct((NC, np_rows, d), jnp.float32),
        mesh=_sc_mesh(),
        compiler_params=pltpu.CompilerParams(use_tc_tiling_on_sc=False),
        scratch_types=[
            pltpu.VMEM((NQA, CHUNK), jnp.int32),
            pltpu.VMEM((NQA, CHUNK), jnp.int32),
            pltpu.VMEM((CHUNK, d), jnp.float32),
            pltpu.VMEM((CHUNK, d), jnp.float32),
            pltpu.VMEM((CHUNK, d), jnp.float32),
            pltpu.VMEM((CHUNK, d), jnp.float32),
            pltpu.VMEM_SHARED((np_rows, d), jnp.float32),
            pltpu.VMEM_SHARED((np_rows, d), jnp.float32),
            pltpu.SemaphoreType.DMA,
            pltpu.SemaphoreType.DMA,
        ],
    )
    def k(tab_hbm, src_hbm, dst_hbm, out_hbm, src_v, dst_v, r0, r1, r2, r3,
          tab_sh, acc_sh, gsem, ssem):
        c = lax.axis_index("c")
        s = lax.axis_index("s")
        wid = c * NS + s
        sl = pl.ds(s * rpt, rpt)
        pltpu.sync_copy(tab_hbm.at[sl], tab_sh.at[sl])
        pltpu.sync_copy(tab_hbm.at[sl], acc_sh.at[sl])
        plsc.subcore_barrier()
        _agg_loop(tab_hbm, tab_sh, src_hbm, dst_hbm, wid, NCH_HALF,
                  src_v, dst_v, [r0, r1, r2, r3], acc_sh, gsem, ssem)
        plsc.subcore_barrier()
        pltpu.sync_copy(acc_sh.at[sl], out_hbm.at[c, sl])

    return k(table, src_r, dst_r)


def _tc_layer1(x, w1, dparts, np_rows):
    n = x.shape[0]
    h = w1.shape[0]
    hh = h // 2

    def body(x_ref, w_ref, d_ref, hs_ref, dis_ref):
        deg = d_ref[0] + d_ref[1] + 1.0
        dis = lax.rsqrt(deg)
        dis_ref[...] = dis
        hraw = lax.dot_general(
            x_ref[...], w_ref[...], (((1,), (1,)), ((), ())),
            preferred_element_type=jnp.float32,
        )
        hs = hraw * dis[:n]
        hs_ref[0, pl.ds(0, n)] = hs[:, :hh]
        hs_ref[1, pl.ds(0, n)] = hs[:, hh:]
        zpad = jnp.zeros((np_rows - n, hh), jnp.float32)
        hs_ref[0, pl.ds(n, np_rows - n)] = zpad
        hs_ref[1, pl.ds(n, np_rows - n)] = zpad

    return pl.pallas_call(
        body,
        out_shape=[
            jax.ShapeDtypeStruct((2, np_rows, hh), jnp.float32),
            jax.ShapeDtypeStruct((np_rows, 1), jnp.float32),
        ],
    )(x, w1, dparts)


def _tc_mid(parts, dis, b1, w2):
    np_rows = parts.shape[1]
    o = w2.shape[0]

    def body(p_ref, dis_ref, b1_ref, w2_ref, hs2_ref):
        agg = jnp.concatenate([p_ref[0], p_ref[1]], axis=1)
        t = jnp.maximum(agg * dis_ref[...] + b1_ref[...], 0.0)
        h2 = lax.dot_general(
            t, w2_ref[...], (((1,), (1,)), ((), ())),
            preferred_element_type=jnp.float32,
        )
        hs2_ref[...] = h2 * dis_ref[...]

    return pl.pallas_call(
        body, out_shape=jax.ShapeDtypeStruct((np_rows, o), jnp.float32)
    )(parts, dis, b1, w2)


def _tc_final(parts, hs2, dis, b2, n):
    o = b2.shape[1]

    def body(q_ref, hs2_ref, dis_ref, b2_ref, o_ref):
        agg = q_ref[0] + q_ref[1] - hs2_ref[...]
        u = (agg * dis_ref[...])[:, :o] + b2_ref[...]
        m = jnp.max(u, axis=1, keepdims=True)
        e = jnp.exp(u - m)
        lse = jnp.log(jnp.sum(e, axis=1, keepdims=True)) + m
        o_ref[...] = (u - lse)[:n]

    return pl.pallas_call(
        body, out_shape=jax.ShapeDtypeStruct((n, o), jnp.float32)
    )(parts, hs2, dis, b2)


@jax.jit
def kernel(x, edge_index, W1, b1, W2, b2):
    n, _ = x.shape
    e = edge_index.shape[1]

    np_rows = (NS * LANES) * math.ceil((n + 1) / (NS * LANES))
    dummy = n

    etot = NCH_ALL * NS * CHUNK
    assert etot >= e and NCH_HALF * NW == NCH_ALL * NS
    src = edge_index[0].astype(jnp.int32)
    dst = edge_index[1].astype(jnp.int32)
    src_p = jnp.concatenate([src, jnp.zeros((etot - e,), jnp.int32)])
    dst_p = jnp.concatenate([dst, jnp.full((etot - e,), dummy, jnp.int32)])
    src_all = src_p.reshape(NS, NCH_ALL, CHUNK)
    dst_all = dst_p.reshape(NS, NCH_ALL, CHUNK)
    src_sym = src_p.reshape(NW, NCH_HALF, CHUNK)
    dst_sym = dst_p.reshape(NW, NCH_HALF, CHUNK)

    dparts = _sc_degree(dst_sym, np_rows)
    hs1_halves, dis = _tc_layer1(x, W1, dparts.reshape(NC, np_rows, 1), np_rows)
    p = _sc_agg_chsplit(hs1_halves, src_all, dst_all, np_rows, W1.shape[0] // 2)
    hs2 = _tc_mid(p, dis, b1.reshape(1, -1), W2)
    q = _sc_agg_edgesplit(hs2, src_sym, dst_sym, np_rows, hs2.shape[1])
    return _tc_final(q, hs2, dis, b2.reshape(1, -1), n)


# speedup vs baseline: 1.0182x; 1.0182x over previous
import functools
import math

import jax
import jax.numpy as jnp
from jax import lax
from jax.experimental import pallas as pl
from jax.experimental.pallas import tpu as pltpu
from jax.experimental.pallas import tpu_sc as plsc

NC = 2
NS = 16
NW = NC * NS
LANES = 16
CHUNK = 128


def _sc_mesh():
    return plsc.VectorSubcoreMesh(
        core_axis_name="c", subcore_axis_name="s", num_cores=NC, num_subcores=NS
    )


def _sc_degree(dst_r, np_rows):
    nch = dst_r.shape[1]
    rpt = np_rows // NS

    @functools.partial(
        pl.kernel,
        out_type=jax.ShapeDtypeStruct((NC, np_rows), jnp.float32),
        mesh=_sc_mesh(),
        scratch_types=[
            pltpu.VMEM((nch, CHUNK), jnp.int32),
            pltpu.VMEM((CHUNK,), jnp.float32),
            pltpu.VMEM((rpt,), jnp.float32),
            pltpu.VMEM_SHARED((np_rows,), jnp.float32),
        ],
    )
    def k(dst_hbm, out_hbm, dst_v, ones_v, z_v, acc_sh):
        c = lax.axis_index("c")
        s = lax.axis_index("s")
        wid = c * NS + s
        pltpu.sync_copy(dst_hbm.at[wid], dst_v)
        for i in range(CHUNK // LANES):
            ones_v[pl.ds(i * LANES, LANES)] = jnp.full((LANES,), 1.0, jnp.float32)
        for i in range(rpt // LANES):
            z_v[pl.ds(i * LANES, LANES)] = jnp.zeros((LANES,), jnp.float32)
        pltpu.sync_copy(z_v, acc_sh.at[pl.ds(s * rpt, rpt)])
        plsc.subcore_barrier()

        def step(j, carry):
            pltpu.sync_copy(ones_v, acc_sh.at[dst_v.at[j]], add=True)
            return carry

        lax.fori_loop(0, nch, step, 0)
        plsc.subcore_barrier()
        pltpu.sync_copy(acc_sh.at[pl.ds(s * rpt, rpt)], out_hbm.at[c, pl.ds(s * rpt, rpt)])

    return k(dst_r)


NCH_ALL = 160
NCH_HALF = 80
NQA = 40


def _agg_loop(tab_dummy_hbm, tab_sh, src_hbm, dst_hbm, tile, nch, src_v, dst_v,
              bufs, acc_sh, gsem, ssem):
    nq4 = NQA // 4
    for seg in range(nch // NQA):
        pltpu.sync_copy(src_hbm.at[tile, pl.ds(seg * NQA, NQA)], src_v)
        pltpu.sync_copy(dst_hbm.at[tile, pl.ds(seg * NQA, NQA)], dst_v)
        pltpu.async_copy(tab_sh.at[src_v.at[0]], bufs[0], gsem)
        pltpu.async_copy(tab_sh.at[src_v.at[1]], bufs[1], gsem)

        def body(j4, carry):
            j0 = j4 * 4
            for b in range(4):
                j = j0 + b
                nb = (b + 2) % 4
                if b < 2:
                    @pl.when(j4 > 0)
                    def _w():
                        pltpu.make_async_copy(
                            tab_dummy_hbm.at[pl.ds(0, CHUNK)], bufs[nb], ssem
                        ).wait()
                else:
                    pltpu.make_async_copy(
                        tab_dummy_hbm.at[pl.ds(0, CHUNK)], bufs[nb], ssem
                    ).wait()
                if b < 2:
                    pltpu.async_copy(tab_sh.at[src_v.at[j + 2]], bufs[nb], gsem)
                else:
                    @pl.when(j4 < nq4 - 1)
                    def _f():
                        pltpu.async_copy(tab_sh.at[src_v.at[j + 2]], bufs[nb], gsem)
                pltpu.make_async_copy(
                    tab_dummy_hbm.at[pl.ds(0, CHUNK)], bufs[b], gsem
                ).wait()
                pltpu.async_copy(bufs[b], acc_sh.at[dst_v.at[j]], ssem, add=True)
            return carry

        lax.fori_loop(0, nq4, body, 0)
        pltpu.make_async_copy(tab_dummy_hbm.at[pl.ds(0, CHUNK)], bufs[2], ssem).wait()
        pltpu.make_async_copy(tab_dummy_hbm.at[pl.ds(0, CHUNK)], bufs[3], ssem).wait()


def _sc_agg_chsplit(tab2, src_r, dst_r, np_rows, d):
    assert src_r.shape == (NS, NCH_ALL, CHUNK)
    rpt = np_rows // NS

    @functools.partial(
        pl.kernel,
        out_type=jax.ShapeDtypeStruct((NC, np_rows, d), jnp.float32),
        mesh=_sc_mesh(),
        compiler_params=pltpu.CompilerParams(use_tc_tiling_on_sc=False),
        scratch_types=[
            pltpu.VMEM((NQA, CHUNK), jnp.int32),
            pltpu.VMEM((NQA, CHUNK), jnp.int32),
            pltpu.VMEM((CHUNK, d), jnp.float32),
            pltpu.VMEM((CHUNK, d), jnp.float32),
            pltpu.VMEM((CHUNK, d), jnp.float32),
            pltpu.VMEM((CHUNK, d), jnp.float32),
            pltpu.VMEM_SHARED((np_rows, d), jnp.float32),
            pltpu.VMEM_SHARED((np_rows, d), jnp.float32),
            pltpu.SemaphoreType.DMA,
            pltpu.SemaphoreType.DMA,
        ],
    )
    def k(tab_hbm, src_hbm, dst_hbm, out_hbm, src_v, dst_v, r0, r1, r2, r3,
          tab_sh, acc_sh, gsem, ssem):
        c = lax.axis_index("c")
        s = lax.axis_index("s")
        sl = pl.ds(s * rpt, rpt)
        pltpu.sync_copy(tab_hbm.at[c, sl], tab_sh.at[sl])
        pltpu.sync_copy(tab_hbm.at[c, sl], acc_sh.at[sl])
        plsc.subcore_barrier()
        _agg_loop(tab_hbm.at[0], tab_sh, src_hbm, dst_hbm, s, NCH_ALL,
                  src_v, dst_v, [r0, r1, r2, r3], acc_sh, gsem, ssem)
        plsc.subcore_barrier()
        pltpu.sync_copy(acc_sh.at[sl], out_hbm.at[c, sl])

    return k(tab2, src_r, dst_r)


def _sc_agg_edgesplit(table, src_r, dst_r, np_rows, d):
    assert src_r.shape == (NW, NCH_HALF, CHUNK)
    rpt = np_rows // NS

    @functools.partial(
        pl.kernel,
        out_type=jax.ShapeDtypeStruct((NC, np_rows, d), jnp.float32),
        mesh=_sc_mesh(),
        compiler_params=pltpu.CompilerParams(use_tc_tiling_on_sc=False),
        scratch_types=[
            pltpu.VMEM((NQA, CHUNK), jnp.int32),
            pltpu.VMEM((NQA, CHUNK), jnp.int32),
            pltpu.VMEM((CHUNK, d), jnp.float32),
            pltpu.VMEM((CHUNK, d), jnp.float32),
            pltpu.VMEM((CHUNK, d), jnp.float32),
            pltpu.VMEM((CHUNK, d), jnp.float32),
            pltpu.VMEM_SHARED((np_rows, d), jnp.float32),
            pltpu.VMEM_SHARED((np_rows, d), jnp.float32),
            pltpu.SemaphoreType.DMA,
            pltpu.SemaphoreType.DMA,
        ],
    )
    def k(tab_hbm, src_hbm, dst_hbm, out_hbm, src_v, dst_v, r0, r1, r2, r3,
          tab_sh, acc_sh, gsem, ssem):
        c = lax.axis_index("c")
        s = lax.axis_index("s")
        wid = c * NS + s
        sl = pl.ds(s * rpt, rpt)
        pltpu.sync_copy(tab_hbm.at[sl], tab_sh.at[sl])
        pltpu.sync_copy(tab_hbm.at[sl], acc_sh.at[sl])
        plsc.subcore_barrier()
        _agg_loop(tab_hbm, tab_sh, src_hbm, dst_hbm, wid, NCH_HALF,
                  src_v, dst_v, [r0, r1, r2, r3], acc_sh, gsem, ssem)
        plsc.subcore_barrier()
        pltpu.sync_copy(acc_sh.at[sl], out_hbm.at[c, sl])

    return k(table, src_r, dst_r)


def _tc_layer1(x, w1, dparts, np_rows):
    n = x.shape[0]
    h = w1.shape[0]
    hh = h // 2

    def body(x_ref, w_ref, d_ref, hs_ref, dis_ref):
        deg = d_ref[0] + d_ref[1] + 1.0
        dis = lax.rsqrt(deg)
        dis_ref[...] = dis
        hraw = lax.dot_general(
            x_ref[...], w_ref[...], (((1,), (1,)), ((), ())),
            preferred_element_type=jnp.float32,
        )
        hs = hraw * dis[:n]
        hs_ref[0, pl.ds(0, n)] = hs[:, :hh]
        hs_ref[1, pl.ds(0, n)] = hs[:, hh:]
        zpad = jnp.zeros((np_rows - n, hh), jnp.float32)
        hs_ref[0, pl.ds(n, np_rows - n)] = zpad
        hs_ref[1, pl.ds(n, np_rows - n)] = zpad

    return pl.pallas_call(
        body,
        out_shape=[
            jax.ShapeDtypeStruct((2, np_rows, hh), jnp.float32),
            jax.ShapeDtypeStruct((np_rows, 1), jnp.float32),
        ],
    )(x, w1, dparts)


def _tc_mid(parts, dis, b1, w2):
    np_rows = parts.shape[1]
    o = w2.shape[0]

    def body(p_ref, dis_ref, b1_ref, w2_ref, hs2_ref):
        agg = jnp.concatenate([p_ref[0], p_ref[1]], axis=1)
        t = jnp.maximum(agg * dis_ref[...] + b1_ref[...], 0.0)
        h2 = lax.dot_general(
            t, w2_ref[...], (((1,), (1,)), ((), ())),
            preferred_element_type=jnp.float32,
        )
        hs2_ref[...] = h2 * dis_ref[...]

    return pl.pallas_call(
        body, out_shape=jax.ShapeDtypeStruct((np_rows, o), jnp.float32)
    )(parts, dis, b1, w2)


def _tc_final(parts, hs2, dis, b2, n):
    o = b2.shape[1]

    def body(q_ref, hs2_ref, dis_ref, b2_ref, o_ref):
        agg = q_ref[0] + q_ref[1] - hs2_ref[...]
        u = (agg * dis_ref[...])[:, :o] + b2_ref[...]
        m = jnp.max(u, axis=1, keepdims=True)
        e = jnp.exp(u - m)
        lse = jnp.log(jnp.sum(e, axis=1, keepdims=True)) + m
        o_ref[...] = (u - lse)[:n]

    return pl.pallas_call(
        body, out_shape=jax.ShapeDtypeStruct((n, o), jnp.float32)
    )(parts, hs2, dis, b2)


@jax.jit
def kernel(x, edge_index, W1, b1, W2, b2):
    n, _ = x.shape
    e = edge_index.shape[1]

    np_rows = (NS * LANES) * math.ceil((n + 1) / (NS * LANES))
    dummy = n

    etot = NCH_ALL * NS * CHUNK
    assert etot >= e and NCH_HALF * NW == NCH_ALL * NS
    src = edge_index[0].astype(jnp.int32)
    dst = edge_index[1].astype(jnp.int32)
    src_p = jnp.concatenate([src, jnp.zeros((etot - e,), jnp.int32)])
    dst_p = jnp.concatenate([dst, jnp.full((etot - e,), dummy, jnp.int32)])
    src_all = src_p.reshape(NS, NCH_ALL, CHUNK)
    dst_all = dst_p.reshape(NS, NCH_ALL, CHUNK)
    src_sym = src_p.reshape(NW, NCH_HALF, CHUNK)
    dst_sym = dst_p.reshape(NW, NCH_HALF, CHUNK)

    dparts = _sc_degree(dst_sym, np_rows)
    hs1_halves, dis = _tc_layer1(x, W1, dparts.reshape(NC, np_rows, 1), np_rows)
    p = _sc_agg_chsplit(hs1_halves, src_all, dst_all, np_rows, W1.shape[0] // 2)
    hs2 = _tc_mid(p, dis, b1.reshape(1, -1), W2)
    q = _sc_agg_edgesplit(hs2, src_sym, dst_sym, np_rows, hs2.shape[1])
    return _tc_final(q, hs2, dis, b2.reshape(1, -1), n)
